# Initial kernel scaffold; baseline (speedup 1.0000x reference)
import jax
import jax.numpy as jnp
from jax.experimental import pallas as pl


def kernel(predictions):
    raise NotImplementedError("write your pallas kernel here")


# speedup vs baseline: 1.0291x; 1.0291x over previous
import jax
import jax.numpy as jnp
from jax.experimental import pallas as pl

TOPK = 1000
THR = 0.25


def _finish_body(g_ref, v_ref, s_ref, b_ref):
    g = g_ref[...]
    v = v_ref[...]
    vs = v * (v > THR)
    cls = g[..., 5:]
    m = cls * vs[..., None]
    s_ref[...] = m * (m > THR)
    xy = g[..., 0:2]
    wh = g[..., 2:4]
    b_ref[...] = jnp.concatenate([xy - wh / 2.0, xy + wh / 2.0], axis=-1)


def kernel(predictions):
    bsz, n, c = predictions.shape
    nc = c - 5
    scores = predictions[..., 4]
    v, idx = jax.lax.top_k(scores, TOPK)
    g = jnp.take_along_axis(predictions, idx[..., None], axis=1)
    s, b = pl.pallas_call(
        _finish_body,
        out_shape=(
            jax.ShapeDtypeStruct((bsz, TOPK, nc), jnp.float32),
            jax.ShapeDtypeStruct((bsz, TOPK, 4), jnp.float32),
        ),
    )(g, v)
    return s, b
